# baseline (speedup 1.0000x reference)
import functools

import jax
import jax.numpy as jnp
from jax import lax
from jax.experimental import pallas as pl
from jax.experimental.pallas import tpu as pltpu
from jax.experimental.pallas import tpu_sc as plsc

f32 = jnp.float32
bf16 = jnp.bfloat16

N = 100000
NP = 100096
E = 1600000
CH = 12800
EP = CH * 128
NW = 32
CPT = CH // NW
CPS = CH // 16
GK = 4
SK = 5
ROWS_G = GK * 128
ROWS_S = SK * 128
NG_G = CPT // GK
NG_FS = CPS // SK
NG_ES = CPT // SK
SLAB = NP // 16
R = 4
RS = 2

_mesh = plsc.VectorSubcoreMesh(core_axis_name="c", subcore_axis_name="s")
_sc_params = pltpu.CompilerParams(use_tc_tiling_on_sc=False)


def _gather_body(a_hbm, b_hbm, dsg, h1, *sc):
    idx = sc[0:R]
    bufa = sc[R:2 * R]
    bufb = sc[2 * R:3 * R]
    gsem = sc[3 * R:4 * R]
    osem = sc[4 * R:5 * R]
    cid = lax.axis_index("c")
    sid = lax.axis_index("s")
    wid = sid * 2 + cid
    base = wid * CPT

    def fire(g, b):
        c0 = base + g * GK
        pltpu.sync_copy(dsg.at[pl.ds(c0, GK)], idx[b])
        for j in range(GK):
            sl = pl.ds(j * 128, 128)
            pltpu.async_copy(a_hbm.at[idx[b].at[j, 0]], bufa[b].at[sl], gsem[b])
            pltpu.async_copy(b_hbm.at[idx[b].at[j, 1]], bufb[b].at[sl], gsem[b])

    def drain_g(b):
        for j in range(GK):
            sl = pl.ds(j * 128, 128)
            pltpu.make_async_copy(a_hbm.at[idx[b].at[j, 0]], bufa[b].at[sl], gsem[b]).wait()
            pltpu.make_async_copy(b_hbm.at[idx[b].at[j, 1]], bufb[b].at[sl], gsem[b]).wait()

    def out_descr(g, b):
        c0 = base + g * GK
        return pltpu.make_async_copy(bufa[b], h1.at[pl.ds(c0 * 128, ROWS_G)], osem[b])

    fire(0, 0)
    fire(1, 1)

    def outer(a, carry):
        for b in range(R):
            g = a * R + b
            b2 = (b + 2) % R

            @pl.when(g >= 2)
            def _():
                out_descr(g - 2, b2).wait()

            @pl.when(g + 2 < NG_G)
            def _():
                fire(g + 2, b2)

            drain_g(b)

            def add_body(i, c):
                for r in range(4):
                    row = i * 4 + r
                    av = bufa[b][row, :]
                    bufa[b][row, :] = av + bufb[b][row, :]
                return c

            lax.fori_loop(0, ROWS_G // 4, add_body, 0)
            out_descr(g, b).start()
        return carry

    lax.fori_loop(0, NG_G // R, outer, 0)
    for g in (NG_G - 2, NG_G - 1):
        out_descr(g, g % R).wait()


_gather_call = functools.partial(
    pl.kernel,
    out_type=jax.ShapeDtypeStruct((EP, 32), bf16),
    mesh=_mesh,
    scratch_types=(
        [pltpu.VMEM((GK, 2, 128), jnp.int32) for _ in range(R)]
        + [pltpu.VMEM((ROWS_G, 32), bf16) for _ in range(2 * R)]
        + [pltpu.SemaphoreType.DMA for _ in range(2 * R)]
    ),
    compiler_params=_sc_params,
)(_gather_body)


def _zero_acc(acc, rows0, sid):
    def zrow(i, c):
        rows0[i, pl.ds(0, 16)] = jnp.zeros((16,), f32)
        return c

    lax.fori_loop(0, ROWS_S, zrow, 0)
    for t in range(9):
        pltpu.sync_copy(rows0, acc.at[pl.ds(sid * SLAB + t * ROWS_S, ROWS_S)])
    rem = SLAB - 9 * ROWS_S
    pltpu.sync_copy(rows0.at[pl.ds(0, rem)],
                    acc.at[pl.ds(sid * SLAB + 9 * ROWS_S, rem)])


def _scatter_ring(h3_at, dsts, acc, idx, rows, lsem, ssem, base, ng):
    def load(g, b):
        c0 = base + g * SK
        pltpu.sync_copy(dsts.at[pl.ds(c0, SK)], idx[b])
        pltpu.async_copy(h3_at(c0), rows[b], lsem[b])

    def rows_descr(g, b):
        c0 = base + g * SK
        return pltpu.make_async_copy(h3_at(c0), rows[b], lsem[b])

    def fire_scatter(b):
        for j in range(SK):
            sl = pl.ds(j * 128, 128)
            pltpu.async_copy(rows[b].at[sl], acc.at[idx[b].at[j]], ssem[b], add=True)

    def drain_scatter(b):
        for j in range(SK):
            sl = pl.ds(j * 128, 128)
            pltpu.make_async_copy(rows[b].at[sl], acc.at[idx[b].at[j]], ssem[b]).wait()

    load(0, 0)

    def outer(a, carry):
        for b in range(RS):
            g = a * RS + b
            b2 = 1 - b

            @pl.when(g >= 1)
            def _():
                drain_scatter(b2)

            @pl.when(g + 1 < ng)
            def _():
                load(g + 1, b2)

            rows_descr(g, b).wait()
            fire_scatter(b)
        return carry

    lax.fori_loop(0, ng // RS, outer, 0)
    drain_scatter((ng - 1) % RS)


def _scatter_fs_body(h3s, dsts, out, acc, *sc):
    idx = sc[0:RS]
    rows = sc[RS:2 * RS]
    lsem = sc[2 * RS:3 * RS]
    ssem = sc[3 * RS:]
    cid = lax.axis_index("c")
    sid = lax.axis_index("s")
    _zero_acc(acc, rows[0], sid)
    plsc.subcore_barrier()

    def h3_at(c0):
        return h3s.at[cid, pl.ds(c0 * 128, ROWS_S)]

    _scatter_ring(h3_at, dsts, acc, idx, rows, lsem, ssem, sid * CPS, NG_FS)
    plsc.subcore_barrier()
    pltpu.sync_copy(acc.at[pl.ds(sid * SLAB, SLAB)],
                    out.at[pl.ds(sid * SLAB, SLAB), pl.ds(cid * 16, 16)])


_scatter_fs_call = functools.partial(
    pl.kernel,
    out_type=jax.ShapeDtypeStruct((NP, 32), f32),
    mesh=_mesh,
    scratch_types=(
        [pltpu.VMEM_SHARED((NP, 16), f32)]
        + [pltpu.VMEM((SK, 128), jnp.int32) for _ in range(RS)]
        + [pltpu.VMEM((ROWS_S, 16), f32) for _ in range(RS)]
        + [pltpu.SemaphoreType.DMA for _ in range(2 * RS)]
    ),
    compiler_params=_sc_params,
)(_scatter_fs_body)


def _scatter_es_body(h3, dsts, out0, out1, acc, *sc):
    idx = sc[0:RS]
    rows = sc[RS:2 * RS]
    lsem = sc[2 * RS:3 * RS]
    ssem = sc[3 * RS:]
    cid = lax.axis_index("c")
    sid = lax.axis_index("s")
    _zero_acc(acc, rows[0], sid)
    plsc.subcore_barrier()
    wid = sid * 2 + cid

    def h3_at(c0):
        return h3.at[pl.ds(c0 * 128, ROWS_S)]

    _scatter_ring(h3_at, dsts, acc, idx, rows, lsem, ssem, wid * CPT, NG_ES)
    plsc.subcore_barrier()
    slab = pl.ds(sid * SLAB, SLAB)

    @pl.when(cid == 0)
    def _():
        pltpu.sync_copy(acc.at[slab], out0.at[slab])

    @pl.when(cid == 1)
    def _():
        pltpu.sync_copy(acc.at[slab], out1.at[slab])


_scatter_es_call = functools.partial(
    pl.kernel,
    out_type=[jax.ShapeDtypeStruct((NP, 16), f32),
              jax.ShapeDtypeStruct((NP, 16), f32)],
    mesh=_mesh,
    scratch_types=(
        [pltpu.VMEM_SHARED((NP, 16), f32)]
        + [pltpu.VMEM((SK, 128), jnp.int32) for _ in range(RS)]
        + [pltpu.VMEM((ROWS_S, 16), f32) for _ in range(RS)]
        + [pltpu.SemaphoreType.DMA for _ in range(2 * RS)]
    ),
    compiler_params=_sc_params,
)(_scatter_es_body)


def _table_tc(x_ref, wd_ref, wb_ref, bd_ref, a_ref, b_ref):
    xv = x_ref[...]
    a_ref[...] = (jnp.dot(xv, wd_ref[...], preferred_element_type=f32)
                  + bd_ref[...]).astype(bf16)
    b_ref[...] = jnp.dot(xv, wb_ref[...], preferred_element_type=f32).astype(bf16)


def _make_table(nrows, in_cols, out_cols, grid):
    blk = nrows // grid
    return pl.pallas_call(
        _table_tc,
        grid=(grid,),
        in_specs=[
            pl.BlockSpec((blk, in_cols), lambda i: (i, 0)),
            pl.BlockSpec((in_cols, out_cols), lambda i: (0, 0)),
            pl.BlockSpec((in_cols, out_cols), lambda i: (0, 0)),
            pl.BlockSpec((1, out_cols), lambda i: (0, 0)),
        ],
        out_specs=[
            pl.BlockSpec((blk, out_cols), lambda i: (i, 0)),
            pl.BlockSpec((blk, out_cols), lambda i: (i, 0)),
        ],
        out_shape=[
            jax.ShapeDtypeStruct((nrows, out_cols), bf16),
            jax.ShapeDtypeStruct((nrows, out_cols), bf16),
        ],
    )


def _mid_split_tc(h_ref, w1_ref, b1_ref, w2_ref, b2_ref, o_ref):
    t = jnp.maximum(h_ref[...].astype(f32), 0.0)
    t = jnp.maximum(jnp.dot(t, w1_ref[...], preferred_element_type=f32) + b1_ref[...], 0.0)
    o = jnp.dot(t, w2_ref[...], preferred_element_type=f32) + b2_ref[...]
    o_ref[0] = o[:, :64]
    o_ref[1] = o[:, 64:]


_mid_split = pl.pallas_call(
    _mid_split_tc,
    grid=(EP // 4 // 1024,),
    in_specs=[
        pl.BlockSpec((1024, 128), lambda i: (i, 0)),
        pl.BlockSpec((128, 128), lambda i: (0, 0)),
        pl.BlockSpec((1, 128), lambda i: (0, 0)),
        pl.BlockSpec((128, 128), lambda i: (0, 0)),
        pl.BlockSpec((1, 128), lambda i: (0, 0)),
    ],
    out_specs=pl.BlockSpec((2, 1024, 64), lambda i: (0, i, 0)),
    out_shape=jax.ShapeDtypeStruct((2, EP // 4, 64), f32),
)


def _mid_tc(h_ref, w1_ref, b1_ref, w2_ref, b2_ref, o_ref):
    t = jnp.maximum(h_ref[...].astype(f32), 0.0)
    t = jnp.maximum(jnp.dot(t, w1_ref[...], preferred_element_type=f32) + b1_ref[...], 0.0)
    o_ref[...] = jnp.dot(t, w2_ref[...], preferred_element_type=f32) + b2_ref[...]


_mid16 = pl.pallas_call(
    _mid_tc,
    grid=(EP // 4 // 1024,),
    in_specs=[
        pl.BlockSpec((1024, 128), lambda i: (i, 0)),
        pl.BlockSpec((128, 128), lambda i: (0, 0)),
        pl.BlockSpec((1, 128), lambda i: (0, 0)),
        pl.BlockSpec((128, 64), lambda i: (0, 0)),
        pl.BlockSpec((1, 64), lambda i: (0, 0)),
    ],
    out_specs=pl.BlockSpec((1024, 64), lambda i: (i, 0)),
    out_shape=jax.ShapeDtypeStruct((EP // 4, 64), f32),
)


def _add_tc(a_ref, b_ref, o_ref):
    o_ref[...] = a_ref[...] + b_ref[...]


_add_call = pl.pallas_call(
    _add_tc,
    grid=(4,),
    in_specs=[
        pl.BlockSpec((3128, 128), lambda i: (i, 0)),
        pl.BlockSpec((3128, 128), lambda i: (i, 0)),
    ],
    out_specs=pl.BlockSpec((3128, 128), lambda i: (i, 0)),
    out_shape=jax.ShapeDtypeStruct((NP * 16 // 128, 128), f32),
)

_table0 = _make_table(NP // 8, 128, 256, 2)
_table12 = _make_table(NP // 4, 128, 128, 2)

_PERM = ([32 * e + f for e in range(4) for f in range(16)]
         + [32 * e + 16 + f for e in range(4) for f in range(16)])


def _prep_first_layer(W, b, fin, copies):
    wa = W[:fin]
    wb = W[fin:]
    eye = jnp.eye(copies, dtype=f32)
    wd_bd = jnp.kron(eye, wa - wb)
    wb_bd = jnp.kron(eye, wb)
    b_t = jnp.tile(b, copies)[None, :]
    return wd_bd, wb_bd, b_t


def _prep_mid(W1, b1, W2, b2, split):
    eye = jnp.eye(4, dtype=f32)
    w1 = jnp.kron(eye, W1)
    bm1 = jnp.tile(b1, 4)[None, :]
    w2 = jnp.kron(eye, W2)
    bm2 = jnp.tile(b2, 4)[None, :]
    if split:
        perm = jnp.array(_PERM, dtype=jnp.int32)
        w2 = w2[:, perm]
        bm2 = bm2[:, perm]
    return w1, bm1, w2, bm2


def kernel(x, pos, edge_index, batch,
           W0_0, b0_0, W0_1, b0_1, W0_2, b0_2,
           W1_0, b1_0, W1_1, b1_1, W1_2, b1_2,
           W2_0, b2_0, W2_1, b2_1, W2_2, b2_2):
    src = edge_index[0]
    dst = edge_index[1]
    pad_e = EP - E
    dst_g = jnp.pad(dst, (0, pad_e)).reshape(CH, 1, 128)
    src_g = jnp.pad(src, (0, pad_e)).reshape(CH, 1, 128)
    dsg = jnp.concatenate([dst_g, src_g], axis=1)
    dst_s = jnp.pad(dst, (0, pad_e), constant_values=N).reshape(CH, 128)
    xp = jnp.pad(x, ((0, NP - N), (0, 0)))

    def block(h_tables, first_w, mid_w, last):
        a_t, b_t = h_tables
        h1 = _gather_call(a_t, b_t, dsg)
        if not last:
            m1, bm1, m2, bm2 = mid_w
            h3s = _mid_split(h1.reshape(EP // 4, 128), m1, bm1, m2, bm2)
            return _scatter_fs_call(h3s.reshape(2, EP, 16), dst_s)
        m1, bm1, m2, bm2 = mid_w
        h3 = _mid16(h1.reshape(EP // 4, 128), m1, bm1, m2, bm2)
        acc0, acc1 = _scatter_es_call(h3.reshape(EP, 16), dst_s)
        out = _add_call(acc0.reshape(NP * 16 // 128, 128),
                        acc1.reshape(NP * 16 // 128, 128))
        return out.reshape(NP, 16)

    wd, wb, bt = _prep_first_layer(W0_0, b0_0, 16, 8)
    a_t, b_t = _table0(xp.reshape(NP // 8, 128), wd, wb, bt)
    h = block((a_t.reshape(NP, 32), b_t.reshape(NP, 32)),
              None, _prep_mid(W0_1, b0_1, W0_2, b0_2, True), False)

    wd, wb, bt = _prep_first_layer(W1_0, b1_0, 32, 4)
    a_t, b_t = _table12(h.reshape(NP // 4, 128), wd, wb, bt)
    h = block((a_t.reshape(NP, 32), b_t.reshape(NP, 32)),
              None, _prep_mid(W1_1, b1_1, W1_2, b1_2, True), False)

    wd, wb, bt = _prep_first_layer(W2_0, b2_0, 32, 4)
    a_t, b_t = _table12(h.reshape(NP // 4, 128), wd, wb, bt)
    out = block((a_t.reshape(NP, 32), b_t.reshape(NP, 32)),
                None, _prep_mid(W2_1, b2_1, W2_2, b2_2, False), True)
    return out[:N]
